# baseline (speedup 1.0000x reference)
import numpy as np
import jax
import jax.numpy as jnp
from jax import lax
from jax.experimental import pallas as pl
from jax.experimental.pallas import tpu as pltpu
from jax.experimental.pallas import tpu_sc as plsc

_L = 16
_T = 1 << 19
_TMASK = _T - 1
_NMIN = 16
_PLS = 1.5
_HID = 64
_P1 = int(np.uint32(2654435761).view(np.int32))
_P2 = 805459861
_RES = [int(np.floor(_NMIN * _PLS ** l)) for l in range(_L)]
_DENSE = [(r + 1) ** 3 <= _T for r in _RES]

_NC, _NS, _LANE = 2, 16, 16
_NW = _NC * _NS
_CH = 512
_NG = _CH // _LANE


def _interleave_body(p0_hbm, p1_hbm, tab3_hbm):
    rows = tab3_hbm.shape[0]
    rw = rows // _NW
    cid = lax.axis_index("c")
    sid = lax.axis_index("s")
    wid = sid * _NC + cid
    nch = 8
    rc = rw // nch

    def body(i, carry):
        r0 = wid * rw + i * rc
        pltpu.sync_copy(p0_hbm.at[pl.ds(r0, rc), :], tab3_hbm.at[pl.ds(r0, rc), 0, :])
        pltpu.sync_copy(p1_hbm.at[pl.ds(r0, rc), :], tab3_hbm.at[pl.ds(r0, rc), 1, :])
        return carry
    lax.fori_loop(0, nch, body, 0)


def _make_interleave(m8):
    mesh = plsc.VectorSubcoreMesh(
        core_axis_name="c", subcore_axis_name="s",
        num_cores=_NC, num_subcores=_NS)
    return pl.kernel(
        _interleave_body,
        out_type=jax.ShapeDtypeStruct((m8, 2, 8), jnp.float32),
        mesh=mesh,
        scratch_types=[],
        compiler_params=pltpu.CompilerParams(
            use_tc_tiling_on_sc=False, needs_layout_passes=False),
    )


def _fire_group(g, l, x_v, w_v, idx_v, off_v, rows_v, tab_hbm, sem):
    res = float(_RES[l])
    b = g * _LANE
    x0 = x_v[0, pl.ds(b, _LANE)]
    x1 = x_v[1, pl.ds(b, _LANE)]
    x2 = x_v[2, pl.ds(b, _LANE)]
    pf0 = x0 * res
    pf1 = x1 * res
    pf2 = x2 * res
    i0 = pf0.astype(jnp.int32)
    i1 = pf1.astype(jnp.int32)
    i2 = pf2.astype(jnp.int32)
    w_v[0, pl.ds(b, _LANE)] = pf0 - i0.astype(jnp.float32)
    w_v[1, pl.ds(b, _LANE)] = pf1 - i1.astype(jnp.float32)
    w_v[2, pl.ds(b, _LANE)] = pf2 - i2.astype(jnp.float32)
    lbase = l * _T
    if _DENSE[l]:
        r1 = _RES[l] + 1
        t00 = i1 + r1 * i2
        tt = {(0, 0): t00, (1, 0): t00 + 1, (0, 1): t00 + r1, (1, 1): t00 + r1 + 1}
        for corner in range(8):
            b0, b1, b2 = corner & 1, (corner >> 1) & 1, (corner >> 2) & 1
            idx = (i0 + r1 * tt[(b1, b2)]) + (b0 + lbase)
            idx_v[g, pl.ds(corner * _LANE, _LANE)] = idx >> 3
            off_v[g, pl.ds(corner * _LANE, _LANE)] = idx & 7
    else:
        h1 = i1 * _P1
        h2 = i2 * _P2
        hh = {}
        for b1 in (0, 1):
            for b2 in (0, 1):
                hh[(b1, b2)] = (h1 + b1 * _P1) ^ (h2 + b2 * _P2)
        for corner in range(8):
            b0, b1, b2 = corner & 1, (corner >> 1) & 1, (corner >> 2) & 1
            idx = (((i0 + b0) ^ hh[(b1, b2)]) & _TMASK) + lbase
            idx_v[g, pl.ds(corner * _LANE, _LANE)] = idx >> 3
            off_v[g, pl.ds(corner * _LANE, _LANE)] = idx & 7
    pltpu.async_copy(tab_hbm.at[idx_v.at[g]], rows_v.at[g], sem)


def _process_group(g, l, w_v, off_v, rows_v, feat_v, lane):
    b = g * _LANE
    w0 = w_v[0, pl.ds(b, _LANE)]
    w1 = w_v[1, pl.ds(b, _LANE)]
    w2 = w_v[2, pl.ds(b, _LANE)]
    u0 = 1.0 - w0
    u1 = 1.0 - w1
    u2 = 1.0 - w2
    ss = {(0, 0): u0 * u1, (1, 0): w0 * u1, (0, 1): u0 * w1, (1, 1): w0 * w1}
    gv = jnp.full((_LANE,), g, jnp.int32)
    acc0 = jnp.zeros((_LANE,), jnp.float32)
    acc1 = jnp.zeros((_LANE,), jnp.float32)
    for corner in range(8):
        b0, b1, b2 = corner & 1, (corner >> 1) & 1, (corner >> 2) & 1
        wt = ss[(b0, b1)] * (w2 if b2 else u2)
        pos = lane + (corner * _LANE)
        off = off_v[g, pl.ds(corner * _LANE, _LANE)]
        f0 = plsc.load_gather(rows_v, [gv, pos, off])
        f1 = plsc.load_gather(rows_v, [gv, pos, off + 8])
        acc0 = acc0 + wt * f0
        acc1 = acc1 + wt * f1
    feat_v[2 * l, pl.ds(b, _LANE)] = acc0
    feat_v[2 * l + 1, pl.ds(b, _LANE)] = acc1


def _encode_body(x0_hbm, x1_hbm, x2_hbm, tab_hbm, feat_hbm,
                 x_v, w_v, idx_v, off_v, rows_v, feat_v, sem):
    n = x0_hbm.shape[0]
    pw = n // _NW
    nchunk = pw // _CH
    cid = lax.axis_index("c")
    sid = lax.axis_index("s")
    wid = sid * _NC + cid
    lane = lax.iota(jnp.int32, _LANE)

    def chunk_body(ci, carry):
        base_pt = wid * pw + ci * _CH
        pltpu.sync_copy(x0_hbm.at[pl.ds(base_pt, _CH)], x_v.at[0])
        pltpu.sync_copy(x1_hbm.at[pl.ds(base_pt, _CH)], x_v.at[1])
        pltpu.sync_copy(x2_hbm.at[pl.ds(base_pt, _CH)], x_v.at[2])
        for l in range(_L):
            def fire_body(g, c, l=l):
                _fire_group(g, l, x_v, w_v, idx_v, off_v, rows_v, tab_hbm, sem)
                return c
            lax.fori_loop(0, _NG, fire_body, 0)

            def drain_body(g, c):
                pltpu.make_async_copy(
                    tab_hbm.at[idx_v.at[0]], rows_v.at[0], sem).wait()
                return c
            lax.fori_loop(0, _NG, drain_body, 0)

            def proc_body(g, c, l=l):
                _process_group(g, l, w_v, off_v, rows_v, feat_v, lane)
                return c
            lax.fori_loop(0, _NG, proc_body, 0)
        blk = wid * nchunk + ci
        pltpu.sync_copy(feat_v, feat_hbm.at[blk])
        return carry

    lax.fori_loop(0, nchunk, chunk_body, 0)


def _make_encode(n):
    nblk = (n // _NW // _CH) * _NW
    mesh = plsc.VectorSubcoreMesh(
        core_axis_name="c", subcore_axis_name="s",
        num_cores=_NC, num_subcores=_NS)
    return pl.kernel(
        _encode_body,
        out_type=jax.ShapeDtypeStruct((nblk, 2 * _L, _CH), jnp.float32),
        mesh=mesh,
        scratch_types=[
            pltpu.VMEM((3, _CH), jnp.float32),
            pltpu.VMEM((3, _CH), jnp.float32),
            pltpu.VMEM((_NG, 8 * _LANE), jnp.int32),
            pltpu.VMEM((_NG, 8 * _LANE), jnp.int32),
            pltpu.VMEM((_NG, 8 * _LANE, 16), jnp.float32),
            pltpu.VMEM((2 * _L, _CH), jnp.float32),
            pltpu.SemaphoreType.DMA,
        ],
        compiler_params=pltpu.CompilerParams(
            use_tc_tiling_on_sc=False, needs_layout_passes=False),
    )


def _mlp_body(ft_ref, w1_ref, w2_ref, w3_ref, o_ref):
    ft = ft_ref[0]
    dn = (((0,), (0,)), ((), ()))
    h = jax.lax.dot_general(w1_ref[...], ft, dn,
                            preferred_element_type=jnp.float32)
    h = jnp.maximum(h, 0.0)
    h = jax.lax.dot_general(w2_ref[...], h, dn,
                            preferred_element_type=jnp.float32)
    h = jnp.maximum(h, 0.0)
    o = jax.lax.dot_general(w3_ref[...], h, dn,
                            preferred_element_type=jnp.float32)
    o_ref[...] = o


def _mlp(feat_blocks, w1, w2, w3):
    nblk = feat_blocks.shape[0]
    n = nblk * _CH
    return pl.pallas_call(
        _mlp_body,
        grid=(nblk,),
        in_specs=[
            pl.BlockSpec((1, 2 * _L, _CH), lambda i: (i, 0, 0)),
            pl.BlockSpec((2 * _L, _HID), lambda i: (0, 0)),
            pl.BlockSpec((_HID, _HID), lambda i: (0, 0)),
            pl.BlockSpec((_HID, 1), lambda i: (0, 0)),
        ],
        out_specs=pl.BlockSpec((1, _CH), lambda i: (0, i)),
        out_shape=jax.ShapeDtypeStruct((1, n), jnp.float32),
    )(feat_blocks, w1, w2, w3)


def kernel(x, table, W1, W2, W3):
    n = x.shape[0]
    x0 = x[:, 0]
    x1 = x[:, 1]
    x2 = x[:, 2]
    m = table.shape[0]
    p0 = table[:, 0].reshape(-1, 8)
    p1 = table[:, 1].reshape(-1, 8)
    tab3 = _make_interleave(m // 8)(p0, p1)
    tab16 = tab3.reshape(-1, 16)
    feat_blocks = _make_encode(n)(x0, x1, x2, tab16)
    out = _mlp(feat_blocks, W1, W2, W3)
    return out.reshape(n, 1)


# speedup vs baseline: 1.3248x; 1.3248x over previous
import numpy as np
import jax
import jax.numpy as jnp
from jax import lax
from jax.experimental import pallas as pl
from jax.experimental.pallas import tpu as pltpu
from jax.experimental.pallas import tpu_sc as plsc

_L = 16
_T = 1 << 19
_TMASK = _T - 1
_NMIN = 16
_PLS = 1.5
_HID = 64
_P1 = int(np.uint32(2654435761).view(np.int32))
_P2 = 805459861
_RES = [int(np.floor(_NMIN * _PLS ** l)) for l in range(_L)]
_DENSE = [(r + 1) ** 3 <= _T for r in _RES]

_NC, _NS, _LANE = 2, 16, 16
_NW = _NC * _NS
_CH = 512
_NG = _CH // _LANE


def _interleave_body(p0_hbm, p1_hbm, tab3_hbm):
    rows = tab3_hbm.shape[0]
    rw = rows // _NW
    cid = lax.axis_index("c")
    sid = lax.axis_index("s")
    wid = sid * _NC + cid
    nch = 8
    rc = rw // nch

    def body(i, carry):
        r0 = wid * rw + i * rc
        pltpu.sync_copy(p0_hbm.at[pl.ds(r0, rc), :], tab3_hbm.at[pl.ds(r0, rc), pl.ds(0, 8)])
        pltpu.sync_copy(p1_hbm.at[pl.ds(r0, rc), :], tab3_hbm.at[pl.ds(r0, rc), pl.ds(8, 8)])
        return carry
    lax.fori_loop(0, nch, body, 0)


def _make_interleave(m8):
    mesh = plsc.VectorSubcoreMesh(
        core_axis_name="c", subcore_axis_name="s",
        num_cores=_NC, num_subcores=_NS)
    return pl.kernel(
        _interleave_body,
        out_type=jax.ShapeDtypeStruct((m8, 16), jnp.float32),
        mesh=mesh,
        scratch_types=[],
        compiler_params=pltpu.CompilerParams(
            use_tc_tiling_on_sc=False, needs_layout_passes=False),
    )


def _fire_group(g, l, x_v, w_v, idx_v, off_v, rows_v, tab_hbm, sem):
    res = float(_RES[l])
    b = g * _LANE
    x0 = x_v[0, pl.ds(b, _LANE)]
    x1 = x_v[1, pl.ds(b, _LANE)]
    x2 = x_v[2, pl.ds(b, _LANE)]
    pf0 = x0 * res
    pf1 = x1 * res
    pf2 = x2 * res
    i0 = pf0.astype(jnp.int32)
    i1 = pf1.astype(jnp.int32)
    i2 = pf2.astype(jnp.int32)
    w_v[0, pl.ds(b, _LANE)] = pf0 - i0.astype(jnp.float32)
    w_v[1, pl.ds(b, _LANE)] = pf1 - i1.astype(jnp.float32)
    w_v[2, pl.ds(b, _LANE)] = pf2 - i2.astype(jnp.float32)
    lbase = l * _T
    if _DENSE[l]:
        r1 = _RES[l] + 1
        t00 = i1 + r1 * i2
        tt = {(0, 0): t00, (1, 0): t00 + 1, (0, 1): t00 + r1, (1, 1): t00 + r1 + 1}
        for corner in range(8):
            b0, b1, b2 = corner & 1, (corner >> 1) & 1, (corner >> 2) & 1
            idx = (i0 + r1 * tt[(b1, b2)]) + (b0 + lbase)
            idx_v[g, pl.ds(corner * _LANE, _LANE)] = idx >> 3
            off_v[g, pl.ds(corner * _LANE, _LANE)] = idx & 7
    else:
        h1 = i1 * _P1
        h2 = i2 * _P2
        hh = {}
        for b1 in (0, 1):
            for b2 in (0, 1):
                hh[(b1, b2)] = (h1 + b1 * _P1) ^ (h2 + b2 * _P2)
        for corner in range(8):
            b0, b1, b2 = corner & 1, (corner >> 1) & 1, (corner >> 2) & 1
            idx = (((i0 + b0) ^ hh[(b1, b2)]) & _TMASK) + lbase
            idx_v[g, pl.ds(corner * _LANE, _LANE)] = idx >> 3
            off_v[g, pl.ds(corner * _LANE, _LANE)] = idx & 7
    pltpu.async_copy(tab_hbm.at[idx_v.at[g]], rows_v.at[g], sem)


def _process_group(g, l, w_v, off_v, rows_v, feat_v, lane):
    b = g * _LANE
    w0 = w_v[0, pl.ds(b, _LANE)]
    w1 = w_v[1, pl.ds(b, _LANE)]
    w2 = w_v[2, pl.ds(b, _LANE)]
    u0 = 1.0 - w0
    u1 = 1.0 - w1
    u2 = 1.0 - w2
    ss = {(0, 0): u0 * u1, (1, 0): w0 * u1, (0, 1): u0 * w1, (1, 1): w0 * w1}
    gv = jnp.full((_LANE,), g, jnp.int32)
    acc0 = jnp.zeros((_LANE,), jnp.float32)
    acc1 = jnp.zeros((_LANE,), jnp.float32)
    for corner in range(8):
        b0, b1, b2 = corner & 1, (corner >> 1) & 1, (corner >> 2) & 1
        wt = ss[(b0, b1)] * (w2 if b2 else u2)
        pos = lane + (corner * _LANE)
        off = off_v[g, pl.ds(corner * _LANE, _LANE)]
        f0 = plsc.load_gather(rows_v, [gv, pos, off])
        f1 = plsc.load_gather(rows_v, [gv, pos, off + 8])
        acc0 = acc0 + wt * f0
        acc1 = acc1 + wt * f1
    feat_v[2 * l, pl.ds(b, _LANE)] = acc0
    feat_v[2 * l + 1, pl.ds(b, _LANE)] = acc1


def _encode_body(x0_hbm, x1_hbm, x2_hbm, tab_hbm, feat_hbm,
                 x_v, w_v, idx_v, off_v, rows_v, feat_v, sem):
    n = x0_hbm.shape[0]
    pw = n // _NW
    nchunk = pw // _CH
    cid = lax.axis_index("c")
    sid = lax.axis_index("s")
    wid = sid * _NC + cid
    lane = lax.iota(jnp.int32, _LANE)

    def chunk_body(ci, carry):
        base_pt = wid * pw + ci * _CH
        pltpu.sync_copy(x0_hbm.at[pl.ds(base_pt, _CH)], x_v.at[0])
        pltpu.sync_copy(x1_hbm.at[pl.ds(base_pt, _CH)], x_v.at[1])
        pltpu.sync_copy(x2_hbm.at[pl.ds(base_pt, _CH)], x_v.at[2])
        for l in range(_L):
            def fire_body(g, c, l=l):
                _fire_group(g, l, x_v, w_v, idx_v, off_v, rows_v, tab_hbm, sem)
                return c
            lax.fori_loop(0, _NG, fire_body, 0)

            def drain_body(g, c):
                pltpu.make_async_copy(
                    tab_hbm.at[idx_v.at[0]], rows_v.at[0], sem).wait()
                return c
            lax.fori_loop(0, _NG, drain_body, 0)

            def proc_body(g, c, l=l):
                _process_group(g, l, w_v, off_v, rows_v, feat_v, lane)
                return c
            lax.fori_loop(0, _NG, proc_body, 0)
        blk = wid * nchunk + ci
        pltpu.sync_copy(feat_v, feat_hbm.at[blk])
        return carry

    lax.fori_loop(0, nchunk, chunk_body, 0)


def _make_encode(n):
    nblk = (n // _NW // _CH) * _NW
    mesh = plsc.VectorSubcoreMesh(
        core_axis_name="c", subcore_axis_name="s",
        num_cores=_NC, num_subcores=_NS)
    return pl.kernel(
        _encode_body,
        out_type=jax.ShapeDtypeStruct((nblk, 2 * _L, _CH), jnp.float32),
        mesh=mesh,
        scratch_types=[
            pltpu.VMEM((3, _CH), jnp.float32),
            pltpu.VMEM((3, _CH), jnp.float32),
            pltpu.VMEM((_NG, 8 * _LANE), jnp.int32),
            pltpu.VMEM((_NG, 8 * _LANE), jnp.int32),
            pltpu.VMEM((_NG, 8 * _LANE, 16), jnp.float32),
            pltpu.VMEM((2 * _L, _CH), jnp.float32),
            pltpu.SemaphoreType.DMA,
        ],
        compiler_params=pltpu.CompilerParams(
            use_tc_tiling_on_sc=False, needs_layout_passes=False),
    )


def _mlp_body(ft_ref, w1_ref, w2_ref, w3_ref, o_ref):
    ft = ft_ref[0]
    dn = (((0,), (0,)), ((), ()))
    h = jax.lax.dot_general(w1_ref[...], ft, dn,
                            preferred_element_type=jnp.float32)
    h = jnp.maximum(h, 0.0)
    h = jax.lax.dot_general(w2_ref[...], h, dn,
                            preferred_element_type=jnp.float32)
    h = jnp.maximum(h, 0.0)
    o = jax.lax.dot_general(w3_ref[...], h, dn,
                            preferred_element_type=jnp.float32)
    o_ref[...] = o


def _mlp(feat_blocks, w1, w2, w3):
    nblk = feat_blocks.shape[0]
    n = nblk * _CH
    return pl.pallas_call(
        _mlp_body,
        grid=(nblk,),
        in_specs=[
            pl.BlockSpec((1, 2 * _L, _CH), lambda i: (i, 0, 0)),
            pl.BlockSpec((2 * _L, _HID), lambda i: (0, 0)),
            pl.BlockSpec((_HID, _HID), lambda i: (0, 0)),
            pl.BlockSpec((_HID, 1), lambda i: (0, 0)),
        ],
        out_specs=pl.BlockSpec((1, _CH), lambda i: (0, i)),
        out_shape=jax.ShapeDtypeStruct((1, n), jnp.float32),
    )(feat_blocks, w1, w2, w3)


def kernel(x, table, W1, W2, W3):
    n = x.shape[0]
    x0 = x[:, 0]
    x1 = x[:, 1]
    x2 = x[:, 2]
    m = table.shape[0]
    p0 = table[:, 0].reshape(-1, 8)
    p1 = table[:, 1].reshape(-1, 8)
    tab16 = _make_interleave(m // 8)(p0, p1)
    feat_blocks = _make_encode(n)(x0, x1, x2, tab16)
    out = _mlp(feat_blocks, W1, W2, W3)
    return out.reshape(n, 1)


# speedup vs baseline: 5.7461x; 4.3374x over previous
import numpy as np
import jax
import jax.numpy as jnp
from jax import lax
from jax.experimental import pallas as pl
from jax.experimental.pallas import tpu as pltpu
from jax.experimental.pallas import tpu_sc as plsc

_L = 16
_T = 1 << 19
_TMASK = _T - 1
_NMIN = 16
_PLS = 1.5
_HID = 64
_P1 = int(np.uint32(2654435761).view(np.int32))
_P2 = 805459861
_RES = [int(np.floor(_NMIN * _PLS ** l)) for l in range(_L)]
_DENSE = [(r + 1) ** 3 <= _T for r in _RES]

_NC, _NS, _LANE = 2, 16, 16
_NW = _NC * _NS
_CH = 512
_NG = _CH // _LANE


_IR = 2048


def _interleave_body(p0_hbm, p1_hbm, tab3_hbm, a0, a1, bb):
    rows = tab3_hbm.shape[0]
    rw = rows // _NW
    cid = lax.axis_index("c")
    sid = lax.axis_index("s")
    wid = sid * _NC + cid
    nch = rw // _IR
    lane = lax.iota(jnp.int32, _LANE)
    r_off = lane >> 3
    c_lo = lane & 7

    def body(i, carry):
        r0 = wid * rw + i * _IR
        pltpu.sync_copy(p0_hbm.at[pl.ds(r0, _IR), :], a0)
        pltpu.sync_copy(p1_hbm.at[pl.ds(r0, _IR), :], a1)

        def rowpair(k, c2):
            row = 2 * k + r_off
            v0 = plsc.load_gather(a0, [row, c_lo])
            v1 = plsc.load_gather(a1, [row, c_lo])
            plsc.store_scatter(bb, [row, c_lo], v0)
            plsc.store_scatter(bb, [row, c_lo + 8], v1)
            return c2
        lax.fori_loop(0, _IR // 2, rowpair, 0)
        pltpu.sync_copy(bb, tab3_hbm.at[pl.ds(r0, _IR), :])
        return carry
    lax.fori_loop(0, nch, body, 0)


def _make_interleave(m8):
    mesh = plsc.VectorSubcoreMesh(
        core_axis_name="c", subcore_axis_name="s",
        num_cores=_NC, num_subcores=_NS)
    return pl.kernel(
        _interleave_body,
        out_type=jax.ShapeDtypeStruct((m8, 16), jnp.float32),
        mesh=mesh,
        scratch_types=[
            pltpu.VMEM((_IR, 8), jnp.float32),
            pltpu.VMEM((_IR, 8), jnp.float32),
            pltpu.VMEM((_IR, 16), jnp.float32),
        ],
        compiler_params=pltpu.CompilerParams(
            use_tc_tiling_on_sc=False, needs_layout_passes=False),
    )


def _fire_group(g, l, x_v, w_v, idx_v, off_v, rows_v, tab_hbm, sem):
    res = float(_RES[l])
    b = g * _LANE
    x0 = x_v[0, pl.ds(b, _LANE)]
    x1 = x_v[1, pl.ds(b, _LANE)]
    x2 = x_v[2, pl.ds(b, _LANE)]
    pf0 = x0 * res
    pf1 = x1 * res
    pf2 = x2 * res
    i0 = pf0.astype(jnp.int32)
    i1 = pf1.astype(jnp.int32)
    i2 = pf2.astype(jnp.int32)
    w_v[0, pl.ds(b, _LANE)] = pf0 - i0.astype(jnp.float32)
    w_v[1, pl.ds(b, _LANE)] = pf1 - i1.astype(jnp.float32)
    w_v[2, pl.ds(b, _LANE)] = pf2 - i2.astype(jnp.float32)
    lbase = l * _T
    if _DENSE[l]:
        r1 = _RES[l] + 1
        t00 = i1 + r1 * i2
        tt = {(0, 0): t00, (1, 0): t00 + 1, (0, 1): t00 + r1, (1, 1): t00 + r1 + 1}
        for corner in range(8):
            b0, b1, b2 = corner & 1, (corner >> 1) & 1, (corner >> 2) & 1
            idx = (i0 + r1 * tt[(b1, b2)]) + (b0 + lbase)
            idx_v[g, pl.ds(corner * _LANE, _LANE)] = idx >> 3
            off_v[g, pl.ds(corner * _LANE, _LANE)] = idx & 7
    else:
        h1 = i1 * _P1
        h2 = i2 * _P2
        hh = {}
        for b1 in (0, 1):
            for b2 in (0, 1):
                hh[(b1, b2)] = (h1 + b1 * _P1) ^ (h2 + b2 * _P2)
        for corner in range(8):
            b0, b1, b2 = corner & 1, (corner >> 1) & 1, (corner >> 2) & 1
            idx = (((i0 + b0) ^ hh[(b1, b2)]) & _TMASK) + lbase
            idx_v[g, pl.ds(corner * _LANE, _LANE)] = idx >> 3
            off_v[g, pl.ds(corner * _LANE, _LANE)] = idx & 7
    pltpu.async_copy(tab_hbm.at[idx_v.at[g]], rows_v.at[g], sem)


def _process_group(g, l, w_v, off_v, rows_v, feat_v, lane):
    b = g * _LANE
    w0 = w_v[0, pl.ds(b, _LANE)]
    w1 = w_v[1, pl.ds(b, _LANE)]
    w2 = w_v[2, pl.ds(b, _LANE)]
    u0 = 1.0 - w0
    u1 = 1.0 - w1
    u2 = 1.0 - w2
    ss = {(0, 0): u0 * u1, (1, 0): w0 * u1, (0, 1): u0 * w1, (1, 1): w0 * w1}
    gv = jnp.full((_LANE,), g, jnp.int32)
    acc0 = jnp.zeros((_LANE,), jnp.float32)
    acc1 = jnp.zeros((_LANE,), jnp.float32)
    for corner in range(8):
        b0, b1, b2 = corner & 1, (corner >> 1) & 1, (corner >> 2) & 1
        wt = ss[(b0, b1)] * (w2 if b2 else u2)
        pos = lane + (corner * _LANE)
        off = off_v[g, pl.ds(corner * _LANE, _LANE)]
        f0 = plsc.load_gather(rows_v, [gv, pos, off])
        f1 = plsc.load_gather(rows_v, [gv, pos, off + 8])
        acc0 = acc0 + wt * f0
        acc1 = acc1 + wt * f1
    feat_v[2 * l, pl.ds(b, _LANE)] = acc0
    feat_v[2 * l + 1, pl.ds(b, _LANE)] = acc1


def _encode_body(x0_hbm, x1_hbm, x2_hbm, tab_hbm, feat_hbm,
                 x_v, w_v, idx_v, off_v, rows_v, feat_v, sem):
    n = x0_hbm.shape[0]
    pw = n // _NW
    nchunk = pw // _CH
    cid = lax.axis_index("c")
    sid = lax.axis_index("s")
    wid = sid * _NC + cid
    lane = lax.iota(jnp.int32, _LANE)

    def chunk_body(ci, carry):
        base_pt = wid * pw + ci * _CH
        pltpu.sync_copy(x0_hbm.at[pl.ds(base_pt, _CH)], x_v.at[0])
        pltpu.sync_copy(x1_hbm.at[pl.ds(base_pt, _CH)], x_v.at[1])
        pltpu.sync_copy(x2_hbm.at[pl.ds(base_pt, _CH)], x_v.at[2])
        for l in range(_L):
            def fire_body(g, c, l=l):
                _fire_group(g, l, x_v, w_v, idx_v, off_v, rows_v, tab_hbm, sem)
                return c
            lax.fori_loop(0, _NG, fire_body, 0)

            def drain_body(g, c):
                pltpu.make_async_copy(
                    tab_hbm.at[idx_v.at[0]], rows_v.at[0], sem).wait()
                return c
            lax.fori_loop(0, _NG, drain_body, 0)

            def proc_body(g, c, l=l):
                _process_group(g, l, w_v, off_v, rows_v, feat_v, lane)
                return c
            lax.fori_loop(0, _NG, proc_body, 0)
        blk = wid * nchunk + ci
        pltpu.sync_copy(feat_v, feat_hbm.at[blk])
        return carry

    lax.fori_loop(0, nchunk, chunk_body, 0)


def _make_encode(n):
    nblk = (n // _NW // _CH) * _NW
    mesh = plsc.VectorSubcoreMesh(
        core_axis_name="c", subcore_axis_name="s",
        num_cores=_NC, num_subcores=_NS)
    return pl.kernel(
        _encode_body,
        out_type=jax.ShapeDtypeStruct((nblk, 2 * _L, _CH), jnp.float32),
        mesh=mesh,
        scratch_types=[
            pltpu.VMEM((3, _CH), jnp.float32),
            pltpu.VMEM((3, _CH), jnp.float32),
            pltpu.VMEM((_NG, 8 * _LANE), jnp.int32),
            pltpu.VMEM((_NG, 8 * _LANE), jnp.int32),
            pltpu.VMEM((_NG, 8 * _LANE, 16), jnp.float32),
            pltpu.VMEM((2 * _L, _CH), jnp.float32),
            pltpu.SemaphoreType.DMA,
        ],
        compiler_params=pltpu.CompilerParams(
            use_tc_tiling_on_sc=False, needs_layout_passes=False),
    )


def _mlp_body(ft_ref, w1_ref, w2_ref, w3_ref, o_ref):
    ft = ft_ref[0]
    dn = (((0,), (0,)), ((), ()))
    h = jax.lax.dot_general(w1_ref[...], ft, dn,
                            preferred_element_type=jnp.float32)
    h = jnp.maximum(h, 0.0)
    h = jax.lax.dot_general(w2_ref[...], h, dn,
                            preferred_element_type=jnp.float32)
    h = jnp.maximum(h, 0.0)
    o = jax.lax.dot_general(w3_ref[...], h, dn,
                            preferred_element_type=jnp.float32)
    o_ref[...] = o


def _mlp(feat_blocks, w1, w2, w3):
    nblk = feat_blocks.shape[0]
    n = nblk * _CH
    return pl.pallas_call(
        _mlp_body,
        grid=(nblk,),
        in_specs=[
            pl.BlockSpec((1, 2 * _L, _CH), lambda i: (i, 0, 0)),
            pl.BlockSpec((2 * _L, _HID), lambda i: (0, 0)),
            pl.BlockSpec((_HID, _HID), lambda i: (0, 0)),
            pl.BlockSpec((_HID, 1), lambda i: (0, 0)),
        ],
        out_specs=pl.BlockSpec((1, _CH), lambda i: (0, i)),
        out_shape=jax.ShapeDtypeStruct((1, n), jnp.float32),
    )(feat_blocks, w1, w2, w3)


def kernel(x, table, W1, W2, W3):
    n = x.shape[0]
    x0 = x[:, 0]
    x1 = x[:, 1]
    x2 = x[:, 2]
    m = table.shape[0]
    p0 = table[:, 0].reshape(-1, 8)
    p1 = table[:, 1].reshape(-1, 8)
    tab16 = _make_interleave(m // 8)(p0, p1)
    feat_blocks = _make_encode(n)(x0, x1, x2, tab16)
    out = _mlp(feat_blocks, W1, W2, W3)
    return out.reshape(n, 1)


# speedup vs baseline: 6.0917x; 1.0602x over previous
import numpy as np
import jax
import jax.numpy as jnp
from jax import lax
from jax.experimental import pallas as pl
from jax.experimental.pallas import tpu as pltpu
from jax.experimental.pallas import tpu_sc as plsc

_L = 16
_T = 1 << 19
_TMASK = _T - 1
_NMIN = 16
_PLS = 1.5
_HID = 64
_P1 = int(np.uint32(2654435761).view(np.int32))
_P2 = 805459861
_RES = [int(np.floor(_NMIN * _PLS ** l)) for l in range(_L)]
_DENSE = [(r + 1) ** 3 <= _T for r in _RES]

_NC, _NS, _LANE = 2, 16, 16
_NW = _NC * _NS
_CH = 256
_NG = _CH // _LANE


_IR = 2048


def _interleave_body(p0_hbm, p1_hbm, tab3_hbm, a0, a1, bb):
    rows = tab3_hbm.shape[0]
    rw = rows // _NW
    cid = lax.axis_index("c")
    sid = lax.axis_index("s")
    wid = sid * _NC + cid
    nch = rw // _IR
    lane = lax.iota(jnp.int32, _LANE)
    r_off = lane >> 3
    c_lo = lane & 7

    def body(i, carry):
        r0 = wid * rw + i * _IR
        pltpu.sync_copy(p0_hbm.at[pl.ds(r0, _IR), :], a0)
        pltpu.sync_copy(p1_hbm.at[pl.ds(r0, _IR), :], a1)

        def rowpair(k, c2):
            row = 2 * k + r_off
            v0 = plsc.load_gather(a0, [row, c_lo])
            v1 = plsc.load_gather(a1, [row, c_lo])
            plsc.store_scatter(bb, [row, c_lo], v0)
            plsc.store_scatter(bb, [row, c_lo + 8], v1)
            return c2
        lax.fori_loop(0, _IR // 2, rowpair, 0)
        pltpu.sync_copy(bb, tab3_hbm.at[pl.ds(r0, _IR), :])
        return carry
    lax.fori_loop(0, nch, body, 0)


def _make_interleave(m8):
    mesh = plsc.VectorSubcoreMesh(
        core_axis_name="c", subcore_axis_name="s",
        num_cores=_NC, num_subcores=_NS)
    return pl.kernel(
        _interleave_body,
        out_type=jax.ShapeDtypeStruct((m8, 16), jnp.float32),
        mesh=mesh,
        scratch_types=[
            pltpu.VMEM((_IR, 8), jnp.float32),
            pltpu.VMEM((_IR, 8), jnp.float32),
            pltpu.VMEM((_IR, 16), jnp.float32),
        ],
        compiler_params=pltpu.CompilerParams(
            use_tc_tiling_on_sc=False, needs_layout_passes=False),
    )


def _fire_group(g, l, x_v, w_v, idx_v, off_v, rows_v, tab_hbm, sem):
    res = float(_RES[l])
    b = g * _LANE
    x0 = x_v[0, pl.ds(b, _LANE)]
    x1 = x_v[1, pl.ds(b, _LANE)]
    x2 = x_v[2, pl.ds(b, _LANE)]
    pf0 = x0 * res
    pf1 = x1 * res
    pf2 = x2 * res
    i0 = pf0.astype(jnp.int32)
    i1 = pf1.astype(jnp.int32)
    i2 = pf2.astype(jnp.int32)
    w_v[0, pl.ds(b, _LANE)] = pf0 - i0.astype(jnp.float32)
    w_v[1, pl.ds(b, _LANE)] = pf1 - i1.astype(jnp.float32)
    w_v[2, pl.ds(b, _LANE)] = pf2 - i2.astype(jnp.float32)
    lbase = l * _T
    if _DENSE[l]:
        r1 = _RES[l] + 1
        t00 = i1 + r1 * i2
        tt = {(0, 0): t00, (1, 0): t00 + 1, (0, 1): t00 + r1, (1, 1): t00 + r1 + 1}
        for corner in range(8):
            b0, b1, b2 = corner & 1, (corner >> 1) & 1, (corner >> 2) & 1
            idx = (i0 + r1 * tt[(b1, b2)]) + (b0 + lbase)
            idx_v[g, pl.ds(corner * _LANE, _LANE)] = idx >> 3
            off_v[g, pl.ds(corner * _LANE, _LANE)] = idx & 7
    else:
        h1 = i1 * _P1
        h2 = i2 * _P2
        hh = {}
        for b1 in (0, 1):
            for b2 in (0, 1):
                hh[(b1, b2)] = (h1 + b1 * _P1) ^ (h2 + b2 * _P2)
        for corner in range(8):
            b0, b1, b2 = corner & 1, (corner >> 1) & 1, (corner >> 2) & 1
            idx = (((i0 + b0) ^ hh[(b1, b2)]) & _TMASK) + lbase
            idx_v[g, pl.ds(corner * _LANE, _LANE)] = idx >> 3
            off_v[g, pl.ds(corner * _LANE, _LANE)] = idx & 7
    pltpu.async_copy(tab_hbm.at[idx_v.at[g]], rows_v.at[g], sem)


def _process_group(g, l, w_v, off_v, rows_v, feat_v, lane):
    b = g * _LANE
    w0 = w_v[0, pl.ds(b, _LANE)]
    w1 = w_v[1, pl.ds(b, _LANE)]
    w2 = w_v[2, pl.ds(b, _LANE)]
    u0 = 1.0 - w0
    u1 = 1.0 - w1
    u2 = 1.0 - w2
    ss = {(0, 0): u0 * u1, (1, 0): w0 * u1, (0, 1): u0 * w1, (1, 1): w0 * w1}
    gv = jnp.full((_LANE,), g, jnp.int32)
    acc0 = jnp.zeros((_LANE,), jnp.float32)
    acc1 = jnp.zeros((_LANE,), jnp.float32)
    for corner in range(8):
        b0, b1, b2 = corner & 1, (corner >> 1) & 1, (corner >> 2) & 1
        wt = ss[(b0, b1)] * (w2 if b2 else u2)
        pos = lane + (corner * _LANE)
        off = off_v[g, pl.ds(corner * _LANE, _LANE)]
        f0 = plsc.load_gather(rows_v, [gv, pos, off])
        f1 = plsc.load_gather(rows_v, [gv, pos, off + 8])
        acc0 = acc0 + wt * f0
        acc1 = acc1 + wt * f1
    feat_v[2 * l, pl.ds(b, _LANE)] = acc0
    feat_v[2 * l + 1, pl.ds(b, _LANE)] = acc1


def _encode_body(x0_hbm, x1_hbm, x2_hbm, tab_hbm, feat_hbm,
                 x_v, w_a, idx_a, off_a, rows_a, w_b, idx_b, off_b, rows_b,
                 feat_v, sem_a, sem_b):
    n = x0_hbm.shape[0]
    pw = n // _NW
    nchunk = pw // _CH
    cid = lax.axis_index("c")
    sid = lax.axis_index("s")
    wid = sid * _NC + cid
    lane = lax.iota(jnp.int32, _LANE)
    bufs = [(w_a, idx_a, off_a, rows_a, sem_a),
            (w_b, idx_b, off_b, rows_b, sem_b)]

    def chunk_body(ci, carry):
        base_pt = wid * pw + ci * _CH
        pltpu.sync_copy(x0_hbm.at[pl.ds(base_pt, _CH)], x_v.at[0])
        pltpu.sync_copy(x1_hbm.at[pl.ds(base_pt, _CH)], x_v.at[1])
        pltpu.sync_copy(x2_hbm.at[pl.ds(base_pt, _CH)], x_v.at[2])

        def pass_a(l):
            w_v, idx_v, off_v, rows_v, sem = bufs[l % 2]

            def fire_body(g, c, l=l):
                _fire_group(g, l, x_v, w_v, idx_v, off_v, rows_v, tab_hbm, sem)
                return c
            lax.fori_loop(0, _NG, fire_body, 0)

        def drain(l):
            w_v, idx_v, off_v, rows_v, sem = bufs[l % 2]

            def drain_body(g, c):
                pltpu.make_async_copy(
                    tab_hbm.at[idx_v.at[0]], rows_v.at[0], sem).wait()
                return c
            lax.fori_loop(0, _NG, drain_body, 0)

        def pass_b(l):
            w_v, idx_v, off_v, rows_v, sem = bufs[l % 2]

            def proc_body(g, c, l=l):
                _process_group(g, l, w_v, off_v, rows_v, feat_v, lane)
                return c
            lax.fori_loop(0, _NG, proc_body, 0)

        pass_a(0)
        for l in range(_L):
            if l + 1 < _L:
                pass_a(l + 1)
            drain(l)
            pass_b(l)
        blk = wid * nchunk + ci
        pltpu.sync_copy(feat_v, feat_hbm.at[blk])
        return carry

    lax.fori_loop(0, nchunk, chunk_body, 0)


def _make_encode(n):
    nblk = (n // _NW // _CH) * _NW
    mesh = plsc.VectorSubcoreMesh(
        core_axis_name="c", subcore_axis_name="s",
        num_cores=_NC, num_subcores=_NS)
    return pl.kernel(
        _encode_body,
        out_type=jax.ShapeDtypeStruct((nblk, 2 * _L, _CH), jnp.float32),
        mesh=mesh,
        scratch_types=[
            pltpu.VMEM((3, _CH), jnp.float32),
            pltpu.VMEM((3, _CH), jnp.float32),
            pltpu.VMEM((_NG, 8 * _LANE), jnp.int32),
            pltpu.VMEM((_NG, 8 * _LANE), jnp.int32),
            pltpu.VMEM((_NG, 8 * _LANE, 16), jnp.float32),
            pltpu.VMEM((3, _CH), jnp.float32),
            pltpu.VMEM((_NG, 8 * _LANE), jnp.int32),
            pltpu.VMEM((_NG, 8 * _LANE), jnp.int32),
            pltpu.VMEM((_NG, 8 * _LANE, 16), jnp.float32),
            pltpu.VMEM((2 * _L, _CH), jnp.float32),
            pltpu.SemaphoreType.DMA,
            pltpu.SemaphoreType.DMA,
        ],
        compiler_params=pltpu.CompilerParams(
            use_tc_tiling_on_sc=False, needs_layout_passes=False),
    )


def _mlp_body(ft_ref, w1_ref, w2_ref, w3_ref, o_ref):
    ft = ft_ref[0]
    dn = (((0,), (0,)), ((), ()))
    h = jax.lax.dot_general(w1_ref[...], ft, dn,
                            preferred_element_type=jnp.float32)
    h = jnp.maximum(h, 0.0)
    h = jax.lax.dot_general(w2_ref[...], h, dn,
                            preferred_element_type=jnp.float32)
    h = jnp.maximum(h, 0.0)
    o = jax.lax.dot_general(w3_ref[...], h, dn,
                            preferred_element_type=jnp.float32)
    o_ref[...] = o


def _mlp(feat_blocks, w1, w2, w3):
    nblk = feat_blocks.shape[0]
    n = nblk * _CH
    return pl.pallas_call(
        _mlp_body,
        grid=(nblk,),
        in_specs=[
            pl.BlockSpec((1, 2 * _L, _CH), lambda i: (i, 0, 0)),
            pl.BlockSpec((2 * _L, _HID), lambda i: (0, 0)),
            pl.BlockSpec((_HID, _HID), lambda i: (0, 0)),
            pl.BlockSpec((_HID, 1), lambda i: (0, 0)),
        ],
        out_specs=pl.BlockSpec((1, _CH), lambda i: (0, i)),
        out_shape=jax.ShapeDtypeStruct((1, n), jnp.float32),
    )(feat_blocks, w1, w2, w3)


def kernel(x, table, W1, W2, W3):
    n = x.shape[0]
    x0 = x[:, 0]
    x1 = x[:, 1]
    x2 = x[:, 2]
    m = table.shape[0]
    p0 = table[:, 0].reshape(-1, 8)
    p1 = table[:, 1].reshape(-1, 8)
    tab16 = _make_interleave(m // 8)(p0, p1)
    feat_blocks = _make_encode(n)(x0, x1, x2, tab16)
    out = _mlp(feat_blocks, W1, W2, W3)
    return out.reshape(n, 1)
